# Initial kernel scaffold; baseline (speedup 1.0000x reference)
import jax
import jax.numpy as jnp
from jax.experimental import pallas as pl


def kernel(x, edge_index, edge_w, Wp1, Wp2, W, b, gamma, beta):
    raise NotImplementedError("write your pallas kernel here")


# speedup vs baseline: 2.1327x; 2.1327x over previous
import jax
import jax.numpy as jnp
from jax import lax
from jax.experimental import pallas as pl
from jax.experimental.pallas import tpu as pltpu
from jax.experimental.pallas import tpu_sc as plsc

N_NODES = 10000
D = 128
DH = D // 2

NC = 2
NS = 16
LANES = 16

CHUNK = 128
NPAD = 10240
ZB = NPAD // NS
ZREP = ZB // CHUNK


def _pos_body(ew_ref, wp1_ref, wp2_ref, out_ref):
    h = jnp.maximum(
        jnp.dot(ew_ref[...], wp1_ref[...], preferred_element_type=jnp.float32), 0.0)
    p = jnp.dot(h, wp2_ref[...], preferred_element_type=jnp.float32) + 1.0
    out_ref[0] = p[:, :DH]
    out_ref[1] = p[:, DH:]


def _pos_call(ew_pad, Wp1, Wp2, blk):
    e_pad = ew_pad.shape[0]
    return pl.pallas_call(
        _pos_body,
        grid=(e_pad // blk,),
        in_specs=[
            pl.BlockSpec((blk, 2), lambda i: (i, 0)),
            pl.BlockSpec((2, D), lambda i: (0, 0)),
            pl.BlockSpec((D, D), lambda i: (0, 0)),
        ],
        out_specs=pl.BlockSpec((NC, blk, DH), lambda i: (0, i, 0)),
        out_shape=jax.ShapeDtypeStruct((NC, e_pad, DH), jnp.float32),
    )(ew_pad, Wp1, Wp2)


def _sc_body(src_hbm, dst_hbm, pos_hbm, x_hbm,
             agg_out, cnt_out,
             src_v, dst_v, xg_v, pos_v,
             ones_v, zrow_v, zcnt_v,
             agg_sh, cnt_sh,
             sem):
    c = lax.axis_index("c")
    s = lax.axis_index("s")

    def _fill(i, _):
        for k in range(DH // LANES):
            zrow_v[i, pl.ds(k * LANES, LANES)] = jnp.zeros((LANES,), jnp.float32)
        ones_v[i, :] = jnp.full((LANES,), 1.0, jnp.float32)
        zcnt_v[i, :] = jnp.zeros((LANES,), jnp.float32)
        return 0
    lax.fori_loop(0, CHUNK, _fill, 0)

    for r in range(ZREP):
        row0 = s * ZB + r * CHUNK
        pltpu.sync_copy(zrow_v, agg_sh.at[pl.ds(row0, CHUNK)])
        pltpu.sync_copy(zcnt_v, cnt_sh.at[pl.ds(row0, CHUNK)])
    plsc.subcore_barrier()

    e_pad = src_hbm.shape[0]
    cpt = e_pad // (CHUNK * NS)
    base0 = s * cpt * CHUNK
    x_c = x_hbm.at[c]

    def _edge_chunk(j, _):
        base = base0 + j * CHUNK
        pltpu.sync_copy(src_hbm.at[pl.ds(base, CHUNK)], src_v)
        pltpu.sync_copy(dst_hbm.at[pl.ds(base, CHUNK)], dst_v)
        pltpu.sync_copy(pos_hbm.at[c, pl.ds(base, CHUNK)], pos_v)
        pltpu.async_copy(x_c.at[src_v], xg_v, sem).wait()

        def _mul_row(i, _):
            for k in range(DH // LANES):
                sl = pl.ds(k * LANES, LANES)
                pos_v[i, sl] = pos_v[i, sl] * xg_v[i, sl]
            return 0
        lax.fori_loop(0, CHUNK, _mul_row, 0)

        pltpu.sync_copy(pos_v, agg_sh.at[dst_v], add=True)

        @pl.when(c == 0)
        def _():
            pltpu.sync_copy(ones_v, cnt_sh.at[dst_v], add=True)
        return 0
    lax.fori_loop(0, cpt, _edge_chunk, 0)

    plsc.subcore_barrier()

    for r in range(ZREP):
        row0 = s * ZB + r * CHUNK
        pltpu.sync_copy(agg_sh.at[pl.ds(row0, CHUNK)],
                        agg_out.at[c, pl.ds(row0, CHUNK)])

        @pl.when(c == 0)
        def _():
            pltpu.sync_copy(cnt_sh.at[pl.ds(row0, CHUNK)],
                            cnt_out.at[pl.ds(row0, CHUNK)])


def _sc_call(src_pad, dst_pad, pos1, x_split):
    mesh = plsc.VectorSubcoreMesh(core_axis_name="c", subcore_axis_name="s")
    f = pl.kernel(
        _sc_body,
        out_type=[
            jax.ShapeDtypeStruct((NC, NPAD, DH), jnp.float32),
            jax.ShapeDtypeStruct((NPAD, LANES), jnp.float32),
        ],
        mesh=mesh,
        compiler_params=pltpu.CompilerParams(use_tc_tiling_on_sc=False),
        scratch_types=[
            pltpu.VMEM((CHUNK,), jnp.int32),
            pltpu.VMEM((CHUNK,), jnp.int32),
            pltpu.VMEM((CHUNK, DH), jnp.float32),
            pltpu.VMEM((CHUNK, DH), jnp.float32),
            pltpu.VMEM((CHUNK, LANES), jnp.float32),
            pltpu.VMEM((CHUNK, DH), jnp.float32),
            pltpu.VMEM((CHUNK, LANES), jnp.float32),
            pltpu.VMEM_SHARED((NPAD, DH), jnp.float32),
            pltpu.VMEM_SHARED((NPAD, LANES), jnp.float32),
            pltpu.SemaphoreType.DMA,
        ],
    )
    return f(src_pad, dst_pad, pos1, x_split)


def _final_body(p_ref, c_ref, x_ref, w_ref, b_ref, g_ref, be_ref, out_ref):
    xv = x_ref[...]
    agg = jnp.concatenate(
        [p_ref[0, :N_NODES, :], p_ref[1, :N_NODES, :]], axis=1) + 2.0 * xv
    cnt = c_ref[:N_NODES, 0:1] + 1.0
    agg = agg / cnt
    o = jnp.dot(agg, w_ref[...], preferred_element_type=jnp.float32) + b_ref[...]
    mean = jnp.mean(o, axis=0, keepdims=True)
    var = jnp.mean((o - mean) * (o - mean), axis=0, keepdims=True)
    o = (o - mean) * lax.rsqrt(var + 1e-5) * g_ref[...] + be_ref[...]
    out_ref[...] = jnp.maximum(o, 0.0) + xv


def _final_call(aggp, cntp, x, W, b, gamma, beta):
    return pl.pallas_call(
        _final_body,
        out_shape=jax.ShapeDtypeStruct((N_NODES, D), jnp.float32),
    )(aggp, cntp, x, W, b.reshape(1, D), gamma.reshape(1, D), beta.reshape(1, D))


def kernel(x, edge_index, edge_w, Wp1, Wp2, W, b, gamma, beta):
    src = edge_index[0]
    dst = edge_index[1]
    e = src.shape[0]
    span = CHUNK * NS
    e_pad = ((e + span - 1) // span) * span

    pad = e_pad - e
    src_p = jnp.concatenate([src, jnp.zeros((pad,), src.dtype)])
    dst_p = jnp.concatenate([dst, jnp.full((pad,), N_NODES, dst.dtype)])
    ew_p = jnp.concatenate([edge_w, jnp.zeros((pad, 2), edge_w.dtype)])
    x_split = jnp.stack([x[:, :DH], x[:, DH:]], axis=0)

    pos1 = _pos_call(ew_p, Wp1, Wp2, blk=1024)
    aggp, cntp = _sc_call(src_p, dst_p, pos1, x_split)
    return _final_call(aggp, cntp, x, W, b, gamma, beta)


# speedup vs baseline: 2.5591x; 1.1999x over previous
import jax
import jax.numpy as jnp
from jax import lax
from jax.experimental import pallas as pl
from jax.experimental.pallas import tpu as pltpu
from jax.experimental.pallas import tpu_sc as plsc

N_NODES = 10000
D = 128
DH = D // 2

NC = 2
NS = 16
LANES = 16

CHUNK = 128
NPAD = 10240
ZB = NPAD // NS
ZREP = ZB // CHUNK


def _pos_body(ew_ref, wp1_ref, wp2_ref, out_ref):
    h = jnp.maximum(
        jnp.dot(ew_ref[...], wp1_ref[...], preferred_element_type=jnp.float32), 0.0)
    p = jnp.dot(h, wp2_ref[...], preferred_element_type=jnp.float32) + 1.0
    out_ref[0] = p[:, :DH]
    out_ref[1] = p[:, DH:]


def _pos_call(ew_pad, Wp1, Wp2, blk):
    e_pad = ew_pad.shape[0]
    return pl.pallas_call(
        _pos_body,
        grid=(e_pad // blk,),
        in_specs=[
            pl.BlockSpec((blk, 2), lambda i: (i, 0)),
            pl.BlockSpec((2, D), lambda i: (0, 0)),
            pl.BlockSpec((D, D), lambda i: (0, 0)),
        ],
        out_specs=pl.BlockSpec((NC, blk, DH), lambda i: (0, i, 0)),
        out_shape=jax.ShapeDtypeStruct((NC, e_pad, DH), jnp.float32),
    )(ew_pad, Wp1, Wp2)


def _sc_body(idx_hbm, pos_hbm, x_hbm,
             agg_out, cnt_out,
             idx0_v, idx1_v, idx2_v, idx3_v,
             pos0_v, pos1_v, xg0_v, xg1_v, msg0_v, msg1_v,
             ones_v, zrow_v, zcnt_v,
             agg_sh, cnt_sh,
             sem_p0, sem_p1, sem_g0, sem_g1,
             sem_a0, sem_a1, sem_c0, sem_c1):
    c = lax.axis_index("c")
    s = lax.axis_index("s")

    idxb = (idx0_v, idx1_v, idx2_v, idx3_v)
    posb = (pos0_v, pos1_v)
    xgb = (xg0_v, xg1_v)
    msgb = (msg0_v, msg1_v)
    sem_p = (sem_p0, sem_p1)
    sem_g = (sem_g0, sem_g1)
    sem_a = (sem_a0, sem_a1)
    sem_c = (sem_c0, sem_c1)

    def _fill(i, _):
        for k in range(DH // LANES):
            zrow_v[i, pl.ds(k * LANES, LANES)] = jnp.zeros((LANES,), jnp.float32)
        ones_v[i, :] = jnp.full((LANES,), 1.0, jnp.float32)
        zcnt_v[i, :] = jnp.zeros((LANES,), jnp.float32)
        return 0
    lax.fori_loop(0, CHUNK, _fill, 0)

    for r in range(ZREP):
        row0 = s * ZB + r * CHUNK
        pltpu.sync_copy(zrow_v, agg_sh.at[pl.ds(row0, CHUNK)])
        pltpu.sync_copy(zcnt_v, cnt_sh.at[pl.ds(row0, CHUNK)])
    plsc.subcore_barrier()

    nch = idx_hbm.shape[0]
    cpt = nch // NS
    base_c = s * cpt
    x_c = x_hbm.at[c]

    def _load(islot, ci):
        pltpu.sync_copy(idx_hbm.at[ci], idxb[islot])
        pltpu.async_copy(pos_hbm.at[c, pl.ds(ci * CHUNK, CHUNK)],
                         posb[islot % 2], sem_p[islot % 2])
        pltpu.async_copy(x_c.at[idxb[islot].at[0]], xgb[islot % 2],
                         sem_g[islot % 2])

    def _wait_in(islot, ci):
        pltpu.make_async_copy(pos_hbm.at[c, pl.ds(ci * CHUNK, CHUNK)],
                              posb[islot % 2], sem_p[islot % 2]).wait()
        pltpu.make_async_copy(x_c.at[idxb[islot].at[0]],
                              xgb[islot % 2], sem_g[islot % 2]).wait()

    def _issue_scatter(islot):
        pltpu.async_copy(msgb[islot % 2], agg_sh.at[idxb[islot].at[1]],
                         sem_a[islot % 2], add=True)

        @pl.when(c == 0)
        def _():
            pltpu.async_copy(ones_v, cnt_sh.at[idxb[islot].at[1]],
                             sem_c[islot % 2], add=True)

    def _wait_scatter(islot):
        pltpu.make_async_copy(msgb[islot % 2], agg_sh.at[idxb[islot].at[1]],
                              sem_a[islot % 2]).wait()

        @pl.when(c == 0)
        def _():
            pltpu.make_async_copy(ones_v, cnt_sh.at[idxb[islot].at[1]],
                                  sem_c[islot % 2]).wait()

    _load(0, base_c)

    def _quad(j0, _):
        for u in range(4):
            j = j0 * 4 + u
            ci = base_c + j
            islot = u
            dslot = u % 2

            if u == 3:
                @pl.when(j0 < cpt // 4 - 1)
                def _():
                    _load(0, ci + 1)
            else:
                _load(u + 1, ci + 1)

            if u < 2:
                @pl.when(j0 > 0)
                def _():
                    _wait_scatter((u + 2) % 4)
            else:
                _wait_scatter(u - 2)

            _wait_in(islot, ci)

            @plsc.parallel_loop(0, CHUNK, step=1, unroll=8)
            def _(i):
                for k in range(DH // LANES):
                    sl = pl.ds(k * LANES, LANES)
                    msgb[dslot][i, sl] = posb[dslot][i, sl] * xgb[dslot][i, sl]

            _issue_scatter(islot)
        return 0
    lax.fori_loop(0, cpt // 4, _quad, 0)

    _wait_scatter(2)
    _wait_scatter(3)
    plsc.subcore_barrier()

    for r in range(ZREP):
        row0 = s * ZB + r * CHUNK
        pltpu.sync_copy(agg_sh.at[pl.ds(row0, CHUNK)],
                        agg_out.at[c, pl.ds(row0, CHUNK)])

        @pl.when(c == 0)
        def _():
            pltpu.sync_copy(cnt_sh.at[pl.ds(row0, CHUNK)],
                            cnt_out.at[pl.ds(row0, CHUNK)])


def _sc_call(idx3, pos1, x_split):
    mesh = plsc.VectorSubcoreMesh(core_axis_name="c", subcore_axis_name="s")
    f = pl.kernel(
        _sc_body,
        out_type=[
            jax.ShapeDtypeStruct((NC, NPAD, DH), jnp.float32),
            jax.ShapeDtypeStruct((NPAD, LANES), jnp.float32),
        ],
        mesh=mesh,
        compiler_params=pltpu.CompilerParams(use_tc_tiling_on_sc=False),
        scratch_types=[
            pltpu.VMEM((2, CHUNK), jnp.int32),
            pltpu.VMEM((2, CHUNK), jnp.int32),
            pltpu.VMEM((2, CHUNK), jnp.int32),
            pltpu.VMEM((2, CHUNK), jnp.int32),
            pltpu.VMEM((CHUNK, DH), jnp.float32),
            pltpu.VMEM((CHUNK, DH), jnp.float32),
            pltpu.VMEM((CHUNK, DH), jnp.float32),
            pltpu.VMEM((CHUNK, DH), jnp.float32),
            pltpu.VMEM((CHUNK, DH), jnp.float32),
            pltpu.VMEM((CHUNK, DH), jnp.float32),
            pltpu.VMEM((CHUNK, LANES), jnp.float32),
            pltpu.VMEM((CHUNK, DH), jnp.float32),
            pltpu.VMEM((CHUNK, LANES), jnp.float32),
            pltpu.VMEM_SHARED((NPAD, DH), jnp.float32),
            pltpu.VMEM_SHARED((NPAD, LANES), jnp.float32),
            pltpu.SemaphoreType.DMA,
            pltpu.SemaphoreType.DMA,
            pltpu.SemaphoreType.DMA,
            pltpu.SemaphoreType.DMA,
            pltpu.SemaphoreType.DMA,
            pltpu.SemaphoreType.DMA,
            pltpu.SemaphoreType.DMA,
            pltpu.SemaphoreType.DMA,
        ],
    )
    return f(idx3, pos1, x_split)


def _final_body(p_ref, c_ref, x_ref, w_ref, b_ref, g_ref, be_ref, out_ref):
    xv = x_ref[...]
    agg = jnp.concatenate(
        [p_ref[0, :N_NODES, :], p_ref[1, :N_NODES, :]], axis=1) + 2.0 * xv
    cnt = c_ref[:N_NODES, 0:1] + 1.0
    agg = agg / cnt
    o = jnp.dot(agg, w_ref[...], preferred_element_type=jnp.float32) + b_ref[...]
    mean = jnp.mean(o, axis=0, keepdims=True)
    var = jnp.mean((o - mean) * (o - mean), axis=0, keepdims=True)
    o = (o - mean) * lax.rsqrt(var + 1e-5) * g_ref[...] + be_ref[...]
    out_ref[...] = jnp.maximum(o, 0.0) + xv


def _final_call(aggp, cntp, x, W, b, gamma, beta):
    return pl.pallas_call(
        _final_body,
        out_shape=jax.ShapeDtypeStruct((N_NODES, D), jnp.float32),
    )(aggp, cntp, x, W, b.reshape(1, D), gamma.reshape(1, D), beta.reshape(1, D))


def kernel(x, edge_index, edge_w, Wp1, Wp2, W, b, gamma, beta):
    src = edge_index[0]
    dst = edge_index[1]
    e = src.shape[0]
    span = CHUNK * NS * 4
    e_pad = ((e + span - 1) // span) * span

    pad = e_pad - e
    src_p = jnp.concatenate([src, jnp.zeros((pad,), src.dtype)])
    dst_p = jnp.concatenate([dst, jnp.full((pad,), N_NODES, dst.dtype)])
    ew_p = jnp.concatenate([edge_w, jnp.zeros((pad, 2), edge_w.dtype)])
    x_split = jnp.stack([x[:, :DH], x[:, DH:]], axis=0)
    idx3 = jnp.stack([src_p.reshape(-1, CHUNK), dst_p.reshape(-1, CHUNK)],
                     axis=1)

    pos1 = _pos_call(ew_p, Wp1, Wp2, blk=1024)
    aggp, cntp = _sc_call(idx3, pos1, x_split)
    return _final_call(aggp, cntp, x, W, b, gamma, beta)


# speedup vs baseline: 4.5211x; 1.7667x over previous
import jax
import jax.numpy as jnp
from jax import lax
from jax.experimental import pallas as pl
from jax.experimental.pallas import tpu as pltpu
from jax.experimental.pallas import tpu_sc as plsc

N_NODES = 10000
D = 128
DH = D // 2

NC = 2
NS = 16
LANES = 16

CHUNK = 128
NPAD = 10240
ZB = NPAD // NS
ZREP = ZB // CHUNK


def _pos_body(ewt_ref, wp1_ref, wp2_ref, out_ref):
    ew = ewt_ref[...].T
    h = jnp.maximum(
        jnp.dot(ew, wp1_ref[...], preferred_element_type=jnp.float32), 0.0)
    out_ref[...] = (
        jnp.dot(h, wp2_ref[...], preferred_element_type=jnp.float32) + 1.0)


def _pos_call(ewt_pad, Wp1, Wp2, blk):
    e_pad = ewt_pad.shape[1]
    return pl.pallas_call(
        _pos_body,
        grid=(e_pad // blk,),
        in_specs=[
            pl.BlockSpec((2, blk), lambda i: (0, i)),
            pl.BlockSpec((2, D), lambda i: (0, 0)),
            pl.BlockSpec((D, D), lambda i: (0, 0)),
        ],
        out_specs=pl.BlockSpec((blk, D), lambda i: (i, 0)),
        out_shape=jax.ShapeDtypeStruct((e_pad, D), jnp.float32),
    )(ewt_pad, Wp1, Wp2)


def _sc_body(idx_hbm, pos_hbm, x_hbm,
             agg_out, cnt_out,
             idx0_v, idx1_v, idx2_v, idx3_v,
             pos0_v, pos1_v, xg0_v, xg1_v, msg0_v, msg1_v,
             ones_v, zrow_v, zcnt_v,
             agg_sh, cnt_sh,
             sem_p0, sem_p1, sem_g0, sem_g1,
             sem_a0, sem_a1, sem_c0, sem_c1):
    c = lax.axis_index("c")
    s = lax.axis_index("s")

    idxb = (idx0_v, idx1_v, idx2_v, idx3_v)
    posb = (pos0_v, pos1_v)
    xgb = (xg0_v, xg1_v)
    msgb = (msg0_v, msg1_v)
    sem_p = (sem_p0, sem_p1)
    sem_g = (sem_g0, sem_g1)
    sem_a = (sem_a0, sem_a1)
    sem_c = (sem_c0, sem_c1)

    def _fill(i, _):
        for k in range(DH // LANES):
            zrow_v[i, pl.ds(k * LANES, LANES)] = jnp.zeros((LANES,), jnp.float32)
        ones_v[i, :] = jnp.full((LANES,), 1.0, jnp.float32)
        zcnt_v[i, :] = jnp.zeros((LANES,), jnp.float32)
        return 0
    lax.fori_loop(0, CHUNK, _fill, 0)

    for r in range(ZREP):
        row0 = s * ZB + r * CHUNK
        pltpu.sync_copy(zrow_v, agg_sh.at[pl.ds(row0, CHUNK)])
        pltpu.sync_copy(zcnt_v, cnt_sh.at[pl.ds(row0, CHUNK)])
    plsc.subcore_barrier()

    nch = idx_hbm.shape[0]
    cpt = nch // NS
    base_c = s * cpt
    x_c = x_hbm.at[c]

    col0 = c * DH

    def _load(islot, ci):
        pltpu.sync_copy(idx_hbm.at[ci], idxb[islot])
        pltpu.async_copy(
            pos_hbm.at[pl.ds(ci * CHUNK, CHUNK), pl.ds(col0, DH)],
            posb[islot % 2], sem_p[islot % 2])
        pltpu.async_copy(x_c.at[idxb[islot].at[0]], xgb[islot % 2],
                         sem_g[islot % 2])

    def _wait_in(islot, ci):
        pltpu.make_async_copy(
            pos_hbm.at[pl.ds(ci * CHUNK, CHUNK), pl.ds(col0, DH)],
            posb[islot % 2], sem_p[islot % 2]).wait()
        pltpu.make_async_copy(x_c.at[idxb[islot].at[0]],
                              xgb[islot % 2], sem_g[islot % 2]).wait()

    def _issue_scatter(islot):
        pltpu.async_copy(msgb[islot % 2], agg_sh.at[idxb[islot].at[1]],
                         sem_a[islot % 2], add=True)

        @pl.when(c == 0)
        def _():
            pltpu.async_copy(ones_v, cnt_sh.at[idxb[islot].at[1]],
                             sem_c[islot % 2], add=True)

    def _wait_scatter(islot):
        pltpu.make_async_copy(msgb[islot % 2], agg_sh.at[idxb[islot].at[1]],
                              sem_a[islot % 2]).wait()

        @pl.when(c == 0)
        def _():
            pltpu.make_async_copy(ones_v, cnt_sh.at[idxb[islot].at[1]],
                                  sem_c[islot % 2]).wait()

    _load(0, base_c)

    def _quad(j0, _):
        for u in range(4):
            j = j0 * 4 + u
            ci = base_c + j
            islot = u
            dslot = u % 2

            if u == 3:
                @pl.when(j0 < cpt // 4 - 1)
                def _():
                    _load(0, ci + 1)
            else:
                _load(u + 1, ci + 1)

            if u < 2:
                @pl.when(j0 > 0)
                def _():
                    _wait_scatter((u + 2) % 4)
            else:
                _wait_scatter(u - 2)

            _wait_in(islot, ci)

            @plsc.parallel_loop(0, CHUNK, step=1, unroll=8)
            def _(i):
                for k in range(DH // LANES):
                    sl = pl.ds(k * LANES, LANES)
                    msgb[dslot][i, sl] = posb[dslot][i, sl] * xgb[dslot][i, sl]

            _issue_scatter(islot)
        return 0
    lax.fori_loop(0, cpt // 4, _quad, 0)

    _wait_scatter(2)
    _wait_scatter(3)
    plsc.subcore_barrier()

    for r in range(ZREP):
        row0 = s * ZB + r * CHUNK
        pltpu.sync_copy(agg_sh.at[pl.ds(row0, CHUNK)],
                        agg_out.at[c, pl.ds(row0, CHUNK)])

        @pl.when(c == 0)
        def _():
            pltpu.sync_copy(cnt_sh.at[pl.ds(row0, CHUNK)],
                            cnt_out.at[pl.ds(row0, CHUNK)])


def _sc_call(idx3, pos1, x_split):
    mesh = plsc.VectorSubcoreMesh(core_axis_name="c", subcore_axis_name="s")
    f = pl.kernel(
        _sc_body,
        out_type=[
            jax.ShapeDtypeStruct((NC, NPAD, DH), jnp.float32),
            jax.ShapeDtypeStruct((NPAD, LANES), jnp.float32),
        ],
        mesh=mesh,
        compiler_params=pltpu.CompilerParams(use_tc_tiling_on_sc=False),
        scratch_types=[
            pltpu.VMEM((2, CHUNK), jnp.int32),
            pltpu.VMEM((2, CHUNK), jnp.int32),
            pltpu.VMEM((2, CHUNK), jnp.int32),
            pltpu.VMEM((2, CHUNK), jnp.int32),
            pltpu.VMEM((CHUNK, DH), jnp.float32),
            pltpu.VMEM((CHUNK, DH), jnp.float32),
            pltpu.VMEM((CHUNK, DH), jnp.float32),
            pltpu.VMEM((CHUNK, DH), jnp.float32),
            pltpu.VMEM((CHUNK, DH), jnp.float32),
            pltpu.VMEM((CHUNK, DH), jnp.float32),
            pltpu.VMEM((CHUNK, LANES), jnp.float32),
            pltpu.VMEM((CHUNK, DH), jnp.float32),
            pltpu.VMEM((CHUNK, LANES), jnp.float32),
            pltpu.VMEM_SHARED((NPAD, DH), jnp.float32),
            pltpu.VMEM_SHARED((NPAD, LANES), jnp.float32),
            pltpu.SemaphoreType.DMA,
            pltpu.SemaphoreType.DMA,
            pltpu.SemaphoreType.DMA,
            pltpu.SemaphoreType.DMA,
            pltpu.SemaphoreType.DMA,
            pltpu.SemaphoreType.DMA,
            pltpu.SemaphoreType.DMA,
            pltpu.SemaphoreType.DMA,
        ],
    )
    return f(idx3, pos1, x_split)


def _final_body(p_ref, c_ref, x_ref, w_ref, b_ref, g_ref, be_ref, out_ref):
    xv = x_ref[...]
    agg = jnp.concatenate(
        [p_ref[0, :N_NODES, :], p_ref[1, :N_NODES, :]], axis=1) + 2.0 * xv
    cnt = c_ref[:N_NODES, 0:1] + 1.0
    agg = agg / cnt
    o = jnp.dot(agg, w_ref[...], preferred_element_type=jnp.float32) + b_ref[...]
    mean = jnp.mean(o, axis=0, keepdims=True)
    var = jnp.mean((o - mean) * (o - mean), axis=0, keepdims=True)
    o = (o - mean) * lax.rsqrt(var + 1e-5) * g_ref[...] + be_ref[...]
    out_ref[...] = jnp.maximum(o, 0.0) + xv


def _final_call(aggp, cntp, x, W, b, gamma, beta):
    return pl.pallas_call(
        _final_body,
        out_shape=jax.ShapeDtypeStruct((N_NODES, D), jnp.float32),
    )(aggp, cntp, x, W, b.reshape(1, D), gamma.reshape(1, D), beta.reshape(1, D))


def kernel(x, edge_index, edge_w, Wp1, Wp2, W, b, gamma, beta):
    src = edge_index[0]
    dst = edge_index[1]
    e = src.shape[0]
    span = CHUNK * NS * 4
    e_pad = ((e + span - 1) // span) * span

    pad = e_pad - e
    src_p = jnp.concatenate([src, jnp.zeros((pad,), src.dtype)])
    dst_p = jnp.concatenate([dst, jnp.full((pad,), N_NODES, dst.dtype)])
    ewt_p = jnp.concatenate(
        [edge_w.T, jnp.zeros((2, pad), edge_w.dtype)], axis=1)
    x_split = jnp.stack([x[:, :DH], x[:, DH:]], axis=0)
    idx3 = jnp.stack([src_p.reshape(-1, CHUNK), dst_p.reshape(-1, CHUNK)],
                     axis=1)

    pos1 = _pos_call(ewt_p, Wp1, Wp2, blk=1024)
    aggp, cntp = _sc_call(idx3, pos1, x_split)
    return _final_call(aggp, cntp, x, W, b, gamma, beta)


# speedup vs baseline: 5.5787x; 1.2339x over previous
import jax
import jax.numpy as jnp
from jax import lax
from jax.experimental import pallas as pl
from jax.experimental.pallas import tpu as pltpu
from jax.experimental.pallas import tpu_sc as plsc

N_NODES = 10000
D = 128
DH = D // 2

NC = 2
NS = 16
LANES = 16

CHUNK = 128
NPAD = 10240
ZB = NPAD // NS
ZREP = ZB // CHUNK


def _pos_body(ewt_ref, wp1_ref, wp2_ref, out_ref):
    ew = ewt_ref[...].T
    h = jnp.maximum(
        jnp.dot(ew, wp1_ref[...], preferred_element_type=jnp.float32), 0.0)
    out_ref[...] = (
        jnp.dot(h, wp2_ref[...], preferred_element_type=jnp.float32) + 1.0)


def _pos_call(ewt_pad, Wp1, Wp2, blk):
    e_pad = ewt_pad.shape[1]
    return pl.pallas_call(
        _pos_body,
        grid=(e_pad // blk,),
        in_specs=[
            pl.BlockSpec((2, blk), lambda i: (0, i)),
            pl.BlockSpec((2, D), lambda i: (0, 0)),
            pl.BlockSpec((D, D), lambda i: (0, 0)),
        ],
        out_specs=pl.BlockSpec((blk, D), lambda i: (i, 0)),
        out_shape=jax.ShapeDtypeStruct((e_pad, D), jnp.float32),
    )(ewt_pad, Wp1, Wp2)


SUP = 8
DW = DH + LANES


def _sc_body(idx_hbm, pos_hbm, x_hbm,
             agg_out,
             isup0_v, isup1_v,
             pos0_v, pos1_v,
             xg0_v, xg1_v, xg2_v, xg3_v,
             msg0_v, msg1_v, agg_sh,
             sem_i0, sem_i1,
             sem_p0, sem_p1,
             sem_g0, sem_g1, sem_g2, sem_g3,
             sem_a0, sem_a1):
    c = lax.axis_index("c")
    s = lax.axis_index("s")
    isupb = (isup0_v, isup1_v)
    posb = (pos0_v, pos1_v)
    xgb = (xg0_v, xg1_v, xg2_v, xg3_v)
    msgb = (msg0_v, msg1_v)
    sem_i = (sem_i0, sem_i1)
    sem_p = (sem_p0, sem_p1)
    sem_g = (sem_g0, sem_g1, sem_g2, sem_g3)
    sem_a = (sem_a0, sem_a1)

    def _fill_zero(i, _):
        for k in range(DW // LANES):
            msg0_v[i, pl.ds(k * LANES, LANES)] = jnp.zeros((LANES,), jnp.float32)
        return 0
    lax.fori_loop(0, CHUNK, _fill_zero, 0)

    for r in range(ZREP):
        row0 = s * ZB + r * CHUNK
        pltpu.sync_copy(msg0_v, agg_sh.at[pl.ds(row0, CHUNK)])

    def _fill_ones(i, _):
        msg0_v[i, pl.ds(DH, LANES)] = jnp.full((LANES,), 1.0, jnp.float32)
        msg1_v[i, pl.ds(DH, LANES)] = jnp.full((LANES,), 1.0, jnp.float32)
        return 0
    lax.fori_loop(0, CHUNK, _fill_ones, 0)
    plsc.subcore_barrier()

    nsup_tot = idx_hbm.shape[0]
    nsup = nsup_tot // NS
    cpt = nsup * SUP
    base_g = s * nsup
    base_c = base_g * SUP
    x_c = x_hbm.at[c]
    col0 = c * DH

    def _load_super(slot, g):
        pltpu.async_copy(idx_hbm.at[g], isupb[slot], sem_i[slot])

    def _wait_super(slot, g):
        pltpu.make_async_copy(idx_hbm.at[g], isupb[slot], sem_i[slot]).wait()

    def _issue_pos(pslot, ci):
        pltpu.async_copy(
            pos_hbm.at[pl.ds(ci * CHUNK, CHUNK), pl.ds(col0, DH)],
            posb[pslot], sem_p[pslot])

    def _wait_pos(pslot, ci):
        pltpu.make_async_copy(
            pos_hbm.at[pl.ds(ci * CHUNK, CHUNK), pl.ds(col0, DH)],
            posb[pslot], sem_p[pslot]).wait()

    def _issue_gather(gslot, src_ref):
        pltpu.async_copy(x_c.at[src_ref], xgb[gslot], sem_g[gslot])

    def _wait_gather(gslot, src_ref):
        pltpu.make_async_copy(x_c.at[src_ref], xgb[gslot], sem_g[gslot]).wait()

    def _issue_scatter(mslot, dst_ref):
        pltpu.async_copy(msgb[mslot], agg_sh.at[dst_ref], sem_a[mslot],
                         add=True)

    def _wait_scatter(mslot, dst_ref):
        pltpu.make_async_copy(msgb[mslot], agg_sh.at[dst_ref],
                              sem_a[mslot]).wait()

    _load_super(0, base_g)
    _wait_super(0, base_g)
    _issue_gather(0, isup0_v.at[0])
    _issue_gather(1, isup0_v.at[1])
    _issue_pos(0, base_c)

    def _spair(g0, _):
        for gg in range(2):
            jbase = (g0 * 2 + gg) * SUP
            sg = gg
            nsg = 1 - gg
            for u in range(SUP):
                j = jbase + u
                ci = base_c + j
                gslot = u % 4
                mslot = u % 2
                src_j = isupb[sg].at[u]
                dst_j = isupb[sg].at[SUP + u]

                if u == 2:
                    @pl.when(j + SUP < cpt)
                    def _():
                        _load_super(nsg, base_g + (g0 * 2 + gg) + 1)
                if u == 6:
                    @pl.when(j + 2 < cpt)
                    def _():
                        _wait_super(nsg, base_g + (g0 * 2 + gg) + 1)

                if u < 6:
                    sref = isupb[sg].at[u + 2]
                else:
                    sref = isupb[nsg].at[u - 6]

                @pl.when(j + 2 < cpt)
                def _():
                    _issue_gather((u + 2) % 4, sref)

                @pl.when(j + 1 < cpt)
                def _():
                    _issue_pos((u + 1) % 2, ci + 1)

                if u < 2:
                    pdst = isupb[nsg].at[SUP + u + 6]

                    @pl.when(j >= 2)
                    def _():
                        _wait_scatter(mslot, pdst)
                else:
                    _wait_scatter(mslot, isupb[sg].at[SUP + u - 2])

                _wait_pos(mslot, ci)
                _wait_gather(gslot, src_j)

                @plsc.parallel_loop(0, CHUNK, step=1, unroll=8)
                def _(i):
                    for k in range(DH // LANES):
                        sl = pl.ds(k * LANES, LANES)
                        msgb[mslot][i, sl] = posb[mslot][i, sl] * xgb[gslot][i, sl]

                _issue_scatter(mslot, dst_j)
        return 0
    lax.fori_loop(0, nsup // 2, _spair, 0)

    _wait_scatter(0, isup1_v.at[SUP + 6])
    _wait_scatter(1, isup1_v.at[SUP + 7])
    plsc.subcore_barrier()

    for r in range(ZREP):
        row0 = s * ZB + r * CHUNK
        pltpu.sync_copy(agg_sh.at[pl.ds(row0, CHUNK)],
                        agg_out.at[c, pl.ds(row0, CHUNK)])


def _sc_call(idx8, pos1, x_split):
    mesh = plsc.VectorSubcoreMesh(core_axis_name="c", subcore_axis_name="s")
    f = pl.kernel(
        _sc_body,
        out_type=jax.ShapeDtypeStruct((NC, NPAD, DW), jnp.float32),
        mesh=mesh,
        compiler_params=pltpu.CompilerParams(use_tc_tiling_on_sc=False),
        scratch_types=(
            [pltpu.VMEM((2 * SUP, CHUNK), jnp.int32)] * 2
            + [pltpu.VMEM((CHUNK, DH), jnp.float32)] * 2
            + [pltpu.VMEM((CHUNK, DH), jnp.float32)] * 4
            + [pltpu.VMEM((CHUNK, DW), jnp.float32)] * 2
            + [pltpu.VMEM_SHARED((NPAD, DW), jnp.float32)]
            + [pltpu.SemaphoreType.DMA] * 10
        ),
    )
    return f(idx8, pos1, x_split)


def _final_body(p_ref, x_ref, w_ref, b_ref, g_ref, be_ref, out_ref):
    xv = x_ref[...]
    agg = jnp.concatenate(
        [p_ref[0, :N_NODES, :DH], p_ref[1, :N_NODES, :DH]], axis=1) + 2.0 * xv
    cnt = p_ref[0, :N_NODES, DH:DH + 1] + 1.0
    agg = agg / cnt
    o = jnp.dot(agg, w_ref[...], preferred_element_type=jnp.float32) + b_ref[...]
    mean = jnp.mean(o, axis=0, keepdims=True)
    var = jnp.mean((o - mean) * (o - mean), axis=0, keepdims=True)
    o = (o - mean) * lax.rsqrt(var + 1e-5) * g_ref[...] + be_ref[...]
    out_ref[...] = jnp.maximum(o, 0.0) + xv


def _final_call(aggp, x, W, b, gamma, beta):
    return pl.pallas_call(
        _final_body,
        out_shape=jax.ShapeDtypeStruct((N_NODES, D), jnp.float32),
    )(aggp, x, W, b.reshape(1, D), gamma.reshape(1, D), beta.reshape(1, D))


def kernel(x, edge_index, edge_w, Wp1, Wp2, W, b, gamma, beta):
    src = edge_index[0]
    dst = edge_index[1]
    e = src.shape[0]
    span = CHUNK * NS * 2 * SUP
    e_pad = ((e + span - 1) // span) * span

    pad = e_pad - e
    src_p = jnp.concatenate([src, jnp.zeros((pad,), src.dtype)])
    dst_p = jnp.concatenate([dst, jnp.full((pad,), N_NODES, dst.dtype)])
    ewt_p = jnp.concatenate(
        [edge_w.T, jnp.zeros((2, pad), edge_w.dtype)], axis=1)
    x_split = jnp.stack([x[:, :DH], x[:, DH:]], axis=0)
    idx8 = jnp.concatenate([src_p.reshape(-1, SUP, CHUNK),
                            dst_p.reshape(-1, SUP, CHUNK)], axis=1)

    pos1 = _pos_call(ewt_p, Wp1, Wp2, blk=4096)
    aggp = _sc_call(idx8, pos1, x_split)
    return _final_call(aggp, x, W, b, gamma, beta)
